# baseline (speedup 1.0000x reference)
import jax
import jax.numpy as jnp
import numpy as np
from jax.experimental import pallas as pl
from jax.experimental.pallas import tpu as pltpu

_T, _V, _F, _H = 50, 64, 8, 64
_TB = 5


def _elu(x):
    return jnp.where(x > 0, x, jnp.exp(x) - 1.0)


def _expansion_consts():
    g = np.arange(_V * _V)
    s, r = g // _V, g % _V
    k = np.arange(_V)
    rep = (s[:, None] == k[None, :]).astype(np.float32)
    til = (r[:, None] == k[None, :]).astype(np.float32)
    rep_tile = np.concatenate([rep, til], axis=1)
    colsum = ((r[None, :] == k[:, None]) & (s != r)[None, :]).astype(np.float32)
    return jnp.asarray(rep_tile), jnp.asarray(colsum)


def _body(x_ref, rt_ref, cs_ref, w1a, b1a, w1b, b1b, w2as, w2ar, b2a, w2b, b2b,
          w3a, b3a, w3b, b3b, w4as, w4ar, w4ak, b4a, w4b, b4b, out_ref):
    f32 = jnp.float32
    dot = lambda a, b: jax.lax.dot(a, b, preferred_element_type=f32)
    vv = _V * _V
    rt = rt_ref[...]
    cs = cs_ref[...]

    x = x_ref[...]
    x1 = _elu(dot(x, w1a[...]) + b1a[...])
    x1 = _elu(dot(x1, w1b[...]) + b1b[...])

    a2 = dot(x1, w2as[...]) + b2a[...]
    b2 = dot(x1, w2ar[...])
    pre2 = jnp.concatenate([
        dot(rt, jnp.concatenate([a2[t * _V:(t + 1) * _V],
                                 b2[t * _V:(t + 1) * _V]], axis=0))
        for t in range(_TB)], axis=0)
    h2 = _elu(pre2)
    x2 = _elu(dot(h2, w2b[...]) + b2b[...])

    agg = jnp.concatenate([
        dot(cs, x2[t * vv:(t + 1) * vv]) for t in range(_TB)], axis=0)

    x3 = _elu(dot(agg, w3a[...]) + b3a[...])
    x3 = _elu(dot(x3, w3b[...]) + b3b[...])

    c4 = dot(x3, w4as[...]) + b4a[...]
    d4 = dot(x3, w4ar[...])
    pre4 = jnp.concatenate([
        dot(rt, jnp.concatenate([c4[t * _V:(t + 1) * _V],
                                 d4[t * _V:(t + 1) * _V]], axis=0))
        for t in range(_TB)], axis=0) + dot(x2, w4ak[...])
    h4 = _elu(pre4)
    o = _elu(dot(h4, w4b[...]) + b4b[...])
    o = o.reshape(_TB, _V, _V, _H)

    jj = jax.lax.broadcasted_iota(jnp.int32, (_TB, _V, _V - 1, _H), 2)
    ss = jax.lax.broadcasted_iota(jnp.int32, (_TB, _V, _V - 1, _H), 1)
    out = jnp.where(jj < ss, o[:, :, :_V - 1, :], o[:, :, 1:, :])
    out_ref[...] = out.reshape(_TB * _V * (_V - 1), _H)


def kernel(inputs, node_masks, all_node_inds, all_graph_info,
           w1a, b1a, w1b, b1b, w2a, b2a, w2b, b2b,
           w3a, b3a, w3b, b3b, w4a, b4a, w4b, b4b):
    b, t, v, f = inputs.shape
    h = w1b.shape[-1]
    x = inputs.reshape(t * v, f) * node_masks.reshape(t * v, 1)
    rep_tile, colsum = _expansion_consts()

    row = lambda z: z.reshape(1, h)
    wspec = lambda s: pl.BlockSpec(s, lambda i: (0, 0))
    args = [
        x, rep_tile, colsum,
        w1a, row(b1a), w1b, row(b1b),
        w2a[:h], w2a[h:], row(b2a), w2b, row(b2b),
        w3a, row(b3a), w3b, row(b3b),
        w4a[:h], w4a[h:2 * h], w4a[2 * h:], row(b4a), w4b, row(b4b),
    ]
    in_specs = [pl.BlockSpec((_TB * v, f), lambda i: (i, 0))]
    in_specs += [wspec(a.shape) for a in args[1:]]

    return pl.pallas_call(
        _body,
        grid=(t // _TB,),
        in_specs=in_specs,
        out_specs=pl.BlockSpec((_TB * v * (v - 1), h), lambda i: (i, 0)),
        out_shape=jax.ShapeDtypeStruct((t * v * (v - 1), h), jnp.float32),
        compiler_params=pltpu.CompilerParams(
            dimension_semantics=("arbitrary",),
        ),
    )(*args)


# speedup vs baseline: 1.5305x; 1.5305x over previous
import jax
import jax.numpy as jnp
from jax.experimental import pallas as pl
from jax.experimental.pallas import tpu as pltpu

_T, _V, _F, _H = 50, 64, 8, 64
_TB = 10
_TP = _TB // 2
_E = _V * (_V - 1)


def _elu(x):
    return jnp.where(x > 0, x, jnp.exp(x) - 1.0)


def _body(x_ref, w1a, b1a, w1b, b1b, w2as, w2ar, b2a, w2b, b2b,
          w3a, b3a, w3b, b3b, w4as, w4ar, w4ak, b4a, w4b, b4b, out_ref):
    f32 = jnp.float32
    dot = lambda a, b: jax.lax.dot(a[...] if isinstance(a, jax.Array) else a[...], b[...],
                                   preferred_element_type=f32)
    h2w = 2 * _H

    x = x_ref[...]
    x1 = _elu(dot(x, w1a) + b1a[...])
    x1 = _elu(dot(x1, w1b) + b1b[...])

    a2 = (dot(x1, w2as) + b2a[...]).reshape(_TP, _V, 1, h2w)
    b2 = dot(x1, w2ar).reshape(_TP, 1, _V, h2w)
    h2 = _elu(a2 + b2)
    x2 = _elu(dot(h2.reshape(_TP * _V * _V, h2w), w2b) + b2b[...])
    g2 = x2.reshape(_TP, _V, _V, h2w)

    s_ids = jax.lax.broadcasted_iota(jnp.int32, (_TP, _V, _V, h2w), 1)
    r_ids = jax.lax.broadcasted_iota(jnp.int32, (_TP, _V, _V, h2w), 2)
    masked = jnp.where(s_ids != r_ids, g2, 0.0)
    agg = jnp.sum(masked, axis=1).reshape(_TP * _V, h2w)

    x3 = _elu(dot(agg, w3a) + b3a[...])
    x3 = _elu(dot(x3, w3b) + b3b[...])

    c4 = (dot(x3, w4as) + b4a[...]).reshape(_TP, _V, 1, h2w)
    d4 = dot(x3, w4ar).reshape(_TP, 1, _V, h2w)
    e4 = dot(x2, w4ak).reshape(_TP, _V, _V, h2w)
    h4 = _elu(c4 + d4 + e4)
    o = _elu(dot(h4.reshape(_TP * _V * _V, h2w), w4b) + b4b[...])
    o = o.reshape(_TP, _V, _V, h2w)

    jj = jax.lax.broadcasted_iota(jnp.int32, (_TP, _V, _V - 1, h2w), 2)
    ss = jax.lax.broadcasted_iota(jnp.int32, (_TP, _V, _V - 1, h2w), 1)
    outp = jnp.where(jj < ss, o[:, :, :_V - 1, :], o[:, :, 1:, :])

    ev = outp[:, :, :, :_H].reshape(_TP * _E, _H)
    od = outp[:, :, :, _H:].reshape(_TP * _E, _H)
    for p in range(_TP):
        out_ref[pl.ds(2 * p * _E, _E), :] = ev[p * _E:(p + 1) * _E]
        out_ref[pl.ds((2 * p + 1) * _E, _E), :] = od[p * _E:(p + 1) * _E]


def _pack2(w):
    z = jnp.zeros_like(w)
    return jnp.concatenate([jnp.concatenate([w, z], axis=1),
                            jnp.concatenate([z, w], axis=1)], axis=0)


def kernel(inputs, node_masks, all_node_inds, all_graph_info,
           w1a, b1a, w1b, b1b, w2a, b2a, w2b, b2b,
           w3a, b3a, w3b, b3b, w4a, b4a, w4b, b4b):
    b, t, v, f = inputs.shape
    h = w1b.shape[-1]
    x = (inputs.reshape(t * v, f) * node_masks.reshape(t * v, 1)).reshape(t, v, f)
    xp = jnp.concatenate([x[0::2], x[1::2]], axis=-1).reshape(t // 2 * v, 2 * f)

    row2 = lambda z: jnp.concatenate([z, z]).reshape(1, 2 * h)
    wspec = lambda s: pl.BlockSpec(s, lambda i: (0, 0))
    args = [
        xp,
        _pack2(w1a), row2(b1a), _pack2(w1b), row2(b1b),
        _pack2(w2a[:h]), _pack2(w2a[h:]), row2(b2a), _pack2(w2b), row2(b2b),
        _pack2(w3a), row2(b3a), _pack2(w3b), row2(b3b),
        _pack2(w4a[:h]), _pack2(w4a[h:2 * h]), _pack2(w4a[2 * h:]), row2(b4a),
        _pack2(w4b), row2(b4b),
    ]
    in_specs = [pl.BlockSpec((_TP * v, 2 * f), lambda i: (i, 0))]
    in_specs += [wspec(a.shape) for a in args[1:]]

    return pl.pallas_call(
        _body,
        grid=(t // _TB,),
        in_specs=in_specs,
        out_specs=pl.BlockSpec((_TB * v * (v - 1), h), lambda i: (i, 0)),
        out_shape=jax.ShapeDtypeStruct((t * v * (v - 1), h), jnp.float32),
        compiler_params=pltpu.CompilerParams(
            dimension_semantics=("arbitrary",),
        ),
    )(*args)
